# baseline (speedup 1.0000x reference)
import functools

import jax
import jax.numpy as jnp
from jax import lax
from jax.experimental import pallas as pl
from jax.experimental.pallas import tpu as pltpu
from jax.experimental.pallas import tpu_sc as plsc

N = 10000
NP = 10240
E = 320000
DIM_H = 512
B = 64
NC = 2
NS = 16
F = 8
FH = 4
NSL = NP // NS
NPS = NP + 4

_EPT = E // NS
_CH = 2000

_mesh = plsc.VectorSubcoreMesh(core_axis_name="c", subcore_axis_name="s")
_sc_params = pltpu.CompilerParams(needs_layout_passes=False)


def _rsqrt16(x):
    i = plsc.bitcast(x, jnp.int32)
    i = jnp.int32(0x5F3759DF) - lax.shift_right_logical(i, 1)
    y = plsc.bitcast(i, jnp.float32)
    for _ in range(3):
        y = y * (1.5 - 0.5 * x * y * y)
    return y


@functools.partial(
    pl.kernel,
    out_type=(
        jax.ShapeDtypeStruct((NC, NS, FH, NP), jnp.float32),
        jax.ShapeDtypeStruct((1, NP), jnp.float32),
    ),
    mesh=_mesh,
    compiler_params=_sc_params,
    scratch_types=[
        pltpu.VMEM((NP,), jnp.float32),
        pltpu.VMEM((NS, NSL), jnp.float32),
        pltpu.VMEM((FH, NPS), jnp.float32),
        pltpu.VMEM((FH, NPS), jnp.float32),
        pltpu.VMEM((_CH,), jnp.int32),
        pltpu.VMEM((_CH,), jnp.int32),
        pltpu.VMEM((_CH,), jnp.float32),
        pltpu.VMEM((FH, NSL), jnp.float32),
        pltpu.VMEM((NSL,), jnp.float32),
        pltpu.VMEM_SHARED((NS, NP), jnp.float32),
        pltpu.VMEM_SHARED((FH, NP), jnp.float32),
    ],
)
def _gcn_sc_kernel(row_hbm, col_hbm, ew_hbm, x8t_hbm, tpart_hbm, d_hbm,
                   degbuf, redbuf, xph, th, rowb, colb, ewb, xbuf, dbuf,
                   shared_deg, shared_xp):
    c = lax.axis_index("c")
    s = lax.axis_index("s")

    scope1 = jax.named_scope("ph1_deg")
    scope1.__enter__()

    @plsc.parallel_loop(0, NP // 16, unroll=8)
    def _(i):
        degbuf[pl.ds(i * 16, 16)] = jnp.zeros((16,), jnp.float32)

    def deg_chunk(ch, carry):
        base = s * _EPT + ch * _CH
        pltpu.sync_copy(col_hbm.at[pl.ds(base, _CH)], colb)
        pltpu.sync_copy(ew_hbm.at[pl.ds(base, _CH)], ewb)

        @plsc.parallel_loop(0, _CH // 16, unroll=8)
        def _(g):
            colv = colb[pl.ds(g * 16, 16)]
            ewv = ewb[pl.ds(g * 16, 16)]
            plsc.addupdate_scatter(degbuf, [colv], ewv)

        return carry

    lax.fori_loop(0, _EPT // _CH, deg_chunk, None)
    scope1.__exit__(None, None, None)

    pltpu.sync_copy(degbuf, shared_deg.at[s])
    plsc.subcore_barrier()

    scope3 = jax.named_scope("ph345_prep")
    scope3.__enter__()
    pltpu.sync_copy(shared_deg.at[:, pl.ds(s * NSL, NSL)], redbuf)

    @plsc.parallel_loop(0, NSL // 16, unroll=2)
    def _(g):
        acc = redbuf[0, pl.ds(g * 16, 16)]
        for slot in range(1, NS):
            acc = acc + redbuf[slot, pl.ds(g * 16, 16)]
        dbuf[pl.ds(g * 16, 16)] = _rsqrt16(acc + 1.0)

    pltpu.sync_copy(
        x8t_hbm.at[pl.ds(c * FH, FH), pl.ds(s * NSL, NSL)], xbuf)

    for r in range(FH):
        @plsc.parallel_loop(0, NSL // 16, unroll=4)
        def _(g, r=r):
            sl = pl.ds(g * 16, 16)
            xbuf[r, sl] = xbuf[r, sl] * dbuf[sl]

    pltpu.sync_copy(xbuf, shared_xp.at[:, pl.ds(s * NSL, NSL)])

    @pl.when(c == 0)
    def _():
        pltpu.sync_copy(dbuf, d_hbm.at[0, pl.ds(s * NSL, NSL)])

    plsc.subcore_barrier()

    pltpu.sync_copy(shared_xp, xph.at[:, pl.ds(0, NP)])

    scope3.__exit__(None, None, None)

    scope6 = jax.named_scope("ph6_agg")
    scope6.__enter__()
    for r in range(FH):
        @plsc.parallel_loop(0, NP // 16, unroll=8)
        def _(i, r=r):
            th[r, pl.ds(i * 16, 16)] = jnp.zeros((16,), jnp.float32)

    iov = lax.iota(jnp.int32, 16)
    io4 = lax.shift_right_logical(iov, 2)
    iom4 = lax.bitwise_and(iov, 3)
    exp_idx = [io4 + 4 * q for q in range(4)]

    def agg_chunk(ch, carry):
        base = s * _EPT + ch * _CH
        pltpu.sync_copy(row_hbm.at[pl.ds(base, _CH)], rowb)
        pltpu.sync_copy(col_hbm.at[pl.ds(base, _CH)], colb)
        pltpu.sync_copy(ew_hbm.at[pl.ds(base, _CH)], ewb)

        @plsc.parallel_loop(0, _CH // 16, unroll=2)
        def _(k):
            rowv = rowb[pl.ds(k * 16, 16)]
            colv = colb[pl.ds(k * 16, 16)]
            ewv = ewb[pl.ds(k * 16, 16)]
            for q in range(4):
                rq = rowv[exp_idx[q]]
                cq = colv[exp_idx[q]]
                eq = ewv[exp_idx[q]]
                xv = plsc.load_gather(xph, [iom4, rq])
                plsc.addupdate_scatter(th, [iom4, cq], xv * eq)

        return carry

    lax.fori_loop(0, _EPT // _CH, agg_chunk, None)
    scope6.__exit__(None, None, None)

    with jax.named_scope("ph7_out"):
        pltpu.sync_copy(th.at[:, pl.ds(0, NP)], tpart_hbm.at[c, s])


_NB = 10
_BN = NP // _NB


def _tail_body(tpart_ref, x8t_ref, d_ref, batch_ref, w1_ref, b1_ref,
               w2_ref, b2_ref, out_ref, acc, cnt):
    i = pl.program_id(0)

    @pl.when(i == 0)
    def _():
        acc[...] = jnp.zeros_like(acc)
        cnt[...] = jnp.zeros_like(cnt)

    tb = jnp.sum(tpart_ref[...], axis=1).reshape(F, _BN)
    dv = d_ref[...]
    sb = dv * (tb + dv * x8t_ref[...])
    hb = lax.dot_general(sb, w1_ref[...], (((0,), (0,)), ((), ())),
                         preferred_element_type=jnp.float32)
    hb = jnp.maximum(hb + b1_ref[...], 0.0)
    bids = batch_ref[...]
    gi = lax.broadcasted_iota(jnp.int32, (B, _BN), 0)
    oh = (gi == bids).astype(jnp.float32)
    acc[...] += lax.dot_general(oh, hb, (((1,), (0,)), ((), ())),
                                preferred_element_type=jnp.float32)
    cnt[...] += jnp.sum(oh, axis=1, keepdims=True)

    @pl.when(i == _NB - 1)
    def _():
        hg = acc[...] / jnp.maximum(cnt[...], 1.0)
        o = lax.dot_general(hg, w2_ref[...], (((1,), (0,)), ((), ())),
                            preferred_element_type=jnp.float32) + b2_ref[...]
        m = jnp.max(o, axis=1, keepdims=True)
        lse = m + jnp.log(jnp.sum(jnp.exp(o - m), axis=1, keepdims=True))
        out_ref[...] = o - lse


def _tail_call(tpart, x8t, d, batch2d, w1p, b1, w2, b2):
    return pl.pallas_call(
        _tail_body,
        grid=(_NB,),
        in_specs=[
            pl.BlockSpec((NC, NS, FH, _BN), lambda i: (0, 0, 0, i)),
            pl.BlockSpec((F, _BN), lambda i: (0, i)),
            pl.BlockSpec((1, _BN), lambda i: (0, i)),
            pl.BlockSpec((1, _BN), lambda i: (0, i)),
            pl.BlockSpec((F, DIM_H), lambda i: (0, 0)),
            pl.BlockSpec((1, DIM_H), lambda i: (0, 0)),
            pl.BlockSpec((DIM_H, 2), lambda i: (0, 0)),
            pl.BlockSpec((1, 2), lambda i: (0, 0)),
        ],
        out_specs=pl.BlockSpec((B, 2), lambda i: (0, 0)),
        out_shape=jax.ShapeDtypeStruct((B, 2), jnp.float32),
        scratch_shapes=[
            pltpu.VMEM((B, DIM_H), jnp.float32),
            pltpu.VMEM((B, 1), jnp.float32),
        ],
    )(tpart, x8t, d, batch2d, w1p, b1, w2, b2)


def kernel(x, edge_index, batch, edge_weight, W1, b1, W2, b2):
    ei = edge_index.astype(jnp.int32)
    row = ei[0]
    col = ei[1]
    ew = edge_weight.astype(jnp.float32)

    batch2d = jnp.concatenate(
        [batch.astype(jnp.int32),
         jnp.full((NP - N,), B, jnp.int32)]).reshape(1, NP)

    x8t = jnp.zeros((F, NP), jnp.float32).at[:7, :N].set(
        x.T.astype(jnp.float32))

    tpart, d = _gcn_sc_kernel(row, col, ew, x8t)

    w1p = jnp.concatenate(
        [W1.astype(jnp.float32), jnp.zeros((1, DIM_H), jnp.float32)], axis=0)
    out = _tail_call(tpart, x8t, d, batch2d, w1p,
                     b1.reshape(1, DIM_H), W2, b2.reshape(1, 2))
    return out


# speedup vs baseline: 1.1106x; 1.1106x over previous
import functools

import jax
import jax.numpy as jnp
from jax import lax
from jax.experimental import pallas as pl
from jax.experimental.pallas import tpu as pltpu
from jax.experimental.pallas import tpu_sc as plsc

N = 10000
NP = 10240
E = 320000
DIM_H = 512
B = 64
NC = 2
NS = 16
F = 8
FH = 4
NSL = NP // NS
NPS = NP + 4

_EPT = E // NS
_CH = 2000

_mesh = plsc.VectorSubcoreMesh(core_axis_name="c", subcore_axis_name="s")
_sc_params = pltpu.CompilerParams(needs_layout_passes=False)


def _rsqrt16(x):
    i = plsc.bitcast(x, jnp.int32)
    i = jnp.int32(0x5F3759DF) - lax.shift_right_logical(i, 1)
    y = plsc.bitcast(i, jnp.float32)
    for _ in range(3):
        y = y * (1.5 - 0.5 * x * y * y)
    return y


@functools.partial(
    pl.kernel,
    out_type=(
        jax.ShapeDtypeStruct((NC, NS, FH, NP), jnp.float32),
        jax.ShapeDtypeStruct((1, NP), jnp.float32),
    ),
    mesh=_mesh,
    compiler_params=_sc_params,
    scratch_types=[
        pltpu.VMEM((NP,), jnp.float32),
        pltpu.VMEM((NS, NSL), jnp.float32),
        [pltpu.VMEM((NP,), jnp.float32) for _ in range(FH)],
        [pltpu.VMEM((NP,), jnp.float32) for _ in range(FH)],
        pltpu.VMEM((_CH,), jnp.int32),
        pltpu.VMEM((_CH,), jnp.int32),
        pltpu.VMEM((_CH,), jnp.float32),
        pltpu.VMEM((FH, NSL), jnp.float32),
        pltpu.VMEM((NSL,), jnp.float32),
        pltpu.VMEM_SHARED((NS, NP), jnp.float32),
        pltpu.VMEM_SHARED((FH, NP), jnp.float32),
    ],
)
def _gcn_sc_kernel(row_hbm, col_hbm, ew_hbm, x8t_hbm, tpart_hbm, d_hbm,
                   deg0, redbuf, xps, ths, rowb, colb, ewb, xbuf, dbuf,
                   shared_deg, shared_xp):
    c = lax.axis_index("c")
    s = lax.axis_index("s")

    scope1 = jax.named_scope("ph1_deg")
    scope1.__enter__()

    @plsc.parallel_loop(0, NP // 16, unroll=8)
    def _(i):
        deg0[pl.ds(i * 16, 16)] = jnp.zeros((16,), jnp.float32)

    def deg_chunk(ch, carry):
        base = s * _EPT + ch * _CH
        pltpu.sync_copy(col_hbm.at[pl.ds(base, _CH)], colb)
        pltpu.sync_copy(ew_hbm.at[pl.ds(base, _CH)], ewb)

        @plsc.parallel_loop(0, _CH // 16, unroll=8)
        def _(g):
            sl0 = pl.ds(g * 16, 16)
            plsc.addupdate_scatter(deg0, [colb[sl0]], ewb[sl0])

        return carry

    lax.fori_loop(0, _EPT // _CH, deg_chunk, None)
    scope1.__exit__(None, None, None)

    pltpu.sync_copy(deg0, shared_deg.at[s])
    plsc.subcore_barrier()

    scope3 = jax.named_scope("ph345_prep")
    scope3.__enter__()
    pltpu.sync_copy(shared_deg.at[:, pl.ds(s * NSL, NSL)], redbuf)

    @plsc.parallel_loop(0, NSL // 16, unroll=2)
    def _(g):
        acc = redbuf[0, pl.ds(g * 16, 16)]
        for slot in range(1, NS):
            acc = acc + redbuf[slot, pl.ds(g * 16, 16)]
        dbuf[pl.ds(g * 16, 16)] = _rsqrt16(acc + 1.0)

    pltpu.sync_copy(
        x8t_hbm.at[pl.ds(c * FH, FH), pl.ds(s * NSL, NSL)], xbuf)

    for r in range(FH):
        @plsc.parallel_loop(0, NSL // 16, unroll=4)
        def _(g, r=r):
            sl = pl.ds(g * 16, 16)
            xbuf[r, sl] = xbuf[r, sl] * dbuf[sl]

    pltpu.sync_copy(xbuf, shared_xp.at[:, pl.ds(s * NSL, NSL)])

    @pl.when(c == 0)
    def _():
        pltpu.sync_copy(dbuf, d_hbm.at[0, pl.ds(s * NSL, NSL)])

    plsc.subcore_barrier()

    for r in range(FH):
        pltpu.sync_copy(shared_xp.at[r], xps[r])

    scope3.__exit__(None, None, None)

    scope6 = jax.named_scope("ph6_agg")
    scope6.__enter__()
    @plsc.parallel_loop(0, NP // 16, unroll=8)
    def _(i):
        sl = pl.ds(i * 16, 16)
        z = jnp.zeros((16,), jnp.float32)
        for r in range(FH):
            ths[r][sl] = z

    def agg_chunk(ch, carry):
        base = s * _EPT + ch * _CH
        pltpu.sync_copy(row_hbm.at[pl.ds(base, _CH)], rowb)
        pltpu.sync_copy(col_hbm.at[pl.ds(base, _CH)], colb)
        pltpu.sync_copy(ew_hbm.at[pl.ds(base, _CH)], ewb)

        @plsc.parallel_loop(0, _CH // 16, unroll=2)
        def _(k):
            sl = pl.ds(k * 16, 16)
            rowv = rowb[sl]
            colv = colb[sl]
            ewv = ewb[sl]
            for r in range(FH):
                xv = plsc.load_gather(xps[r], [rowv])
                plsc.addupdate_scatter(ths[r], [colv], xv * ewv)

        return carry

    lax.fori_loop(0, _EPT // _CH, agg_chunk, None)
    scope6.__exit__(None, None, None)

    with jax.named_scope("ph7_out"):
        for r in range(FH):
            pltpu.sync_copy(ths[r], tpart_hbm.at[c, s, r])


_NB = 10
_BN = NP // _NB


def _tail_body(tpart_ref, x8t_ref, d_ref, batch_ref, w1_ref, b1_ref,
               w2_ref, b2_ref, out_ref, acc, cnt):
    i = pl.program_id(0)

    @pl.when(i == 0)
    def _():
        acc[...] = jnp.zeros_like(acc)
        cnt[...] = jnp.zeros_like(cnt)

    tb = jnp.sum(tpart_ref[...], axis=1).reshape(F, _BN)
    dv = d_ref[...]
    sb = dv * (tb + dv * x8t_ref[...])
    hb = lax.dot_general(sb, w1_ref[...], (((0,), (0,)), ((), ())),
                         preferred_element_type=jnp.float32)
    hb = jnp.maximum(hb + b1_ref[...], 0.0)
    bids = batch_ref[...]
    gi = lax.broadcasted_iota(jnp.int32, (B, _BN), 0)
    oh = (gi == bids).astype(jnp.float32)
    acc[...] += lax.dot_general(oh, hb, (((1,), (0,)), ((), ())),
                                preferred_element_type=jnp.float32)
    cnt[...] += jnp.sum(oh, axis=1, keepdims=True)

    @pl.when(i == _NB - 1)
    def _():
        hg = acc[...] / jnp.maximum(cnt[...], 1.0)
        o = lax.dot_general(hg, w2_ref[...], (((1,), (0,)), ((), ())),
                            preferred_element_type=jnp.float32) + b2_ref[...]
        m = jnp.max(o, axis=1, keepdims=True)
        lse = m + jnp.log(jnp.sum(jnp.exp(o - m), axis=1, keepdims=True))
        out_ref[...] = o - lse


def _tail_call(tpart, x8t, d, batch2d, w1p, b1, w2, b2):
    return pl.pallas_call(
        _tail_body,
        grid=(_NB,),
        in_specs=[
            pl.BlockSpec((NC, NS, FH, _BN), lambda i: (0, 0, 0, i)),
            pl.BlockSpec((F, _BN), lambda i: (0, i)),
            pl.BlockSpec((1, _BN), lambda i: (0, i)),
            pl.BlockSpec((1, _BN), lambda i: (0, i)),
            pl.BlockSpec((F, DIM_H), lambda i: (0, 0)),
            pl.BlockSpec((1, DIM_H), lambda i: (0, 0)),
            pl.BlockSpec((DIM_H, 2), lambda i: (0, 0)),
            pl.BlockSpec((1, 2), lambda i: (0, 0)),
        ],
        out_specs=pl.BlockSpec((B, 2), lambda i: (0, 0)),
        out_shape=jax.ShapeDtypeStruct((B, 2), jnp.float32),
        scratch_shapes=[
            pltpu.VMEM((B, DIM_H), jnp.float32),
            pltpu.VMEM((B, 1), jnp.float32),
        ],
    )(tpart, x8t, d, batch2d, w1p, b1, w2, b2)


def kernel(x, edge_index, batch, edge_weight, W1, b1, W2, b2):
    ei = edge_index.astype(jnp.int32)
    row = ei[0]
    col = ei[1]
    ew = edge_weight.astype(jnp.float32)

    batch2d = jnp.concatenate(
        [batch.astype(jnp.int32),
         jnp.full((NP - N,), B, jnp.int32)]).reshape(1, NP)

    x8t = jnp.zeros((F, NP), jnp.float32).at[:7, :N].set(
        x.T.astype(jnp.float32))

    tpart, d = _gcn_sc_kernel(row, col, ew, x8t)

    w1p = jnp.concatenate(
        [W1.astype(jnp.float32), jnp.zeros((1, DIM_H), jnp.float32)], axis=0)
    out = _tail_call(tpart, x8t, d, batch2d, w1p,
                     b1.reshape(1, DIM_H), W2, b2.reshape(1, 2))
    return out
